# baseline (speedup 1.0000x reference)
import functools

import jax
import jax.numpy as jnp
from jax import lax
from jax.experimental import pallas as pl
from jax.experimental.pallas import tpu as pltpu
from jax.experimental.pallas import tpu_sc as plsc

N = 10000
F = 128
C = 10
NB = 8
NC, NS = 2, 16
NW = NC * NS
CHUNK = 128
NPAD1 = 10240
ZCH1 = NPAD1 // NS
NPAD2 = 10112
ZROWS = NPAD2 // NS
E1, E2 = 320000, 160000
_GRAN = NW * CHUNK * 3
E1P = ((E1 + _GRAN - 1) // _GRAN) * _GRAN
E2P = ((E2 + _GRAN - 1) // _GRAN) * _GRAN
N1 = E1P // (NW * CHUNK)
N2 = E2P // (NW * CHUNK)
IB1 = 9
IB2 = 6

_mesh = plsc.VectorSubcoreMesh(core_axis_name="c", subcore_axis_name="s",
                               num_cores=NC, num_subcores=NS)


def _deg_body(dpk1, dpk2, zflat, degp, acc1, acc2, evd1, evd2, ones_v, ztmp):
    c = lax.axis_index("c")
    s = lax.axis_index("s")
    w = c * NS + s
    pltpu.sync_copy(zflat, ztmp)
    pltpu.sync_copy(ztmp, acc1.at[pl.ds(s * ZCH1, ZCH1)])
    pltpu.sync_copy(ztmp, acc2.at[pl.ds(s * ZCH1, ZCH1)])
    for j in range(CHUNK // 16):
        ones_v[pl.ds(j * 16, 16)] = jnp.ones((16,), jnp.float32)
    plsc.subcore_barrier()

    def run_list(dpk, acc, ni, ib, evd):
        ngr = ni // ib
        gbase = w * ngr

        def body(g, carry):
            pltpu.sync_copy(dpk.at[gbase + g], evd)
            for j in range(ib):
                pltpu.sync_copy(ones_v, acc.at[evd.at[j]], add=True)
            return carry

        lax.fori_loop(0, ngr, body, 0)

    run_list(dpk1, acc1, N1, IB1, evd1)
    run_list(dpk2, acc2, N2, IB2, evd2)
    plsc.subcore_barrier()
    pltpu.sync_copy(acc1.at[pl.ds(s * ZCH1, ZCH1)], ztmp)
    pltpu.sync_copy(ztmp, degp.at[pl.ds((c * 2 + 0) * NPAD1 + s * ZCH1, ZCH1)])
    pltpu.sync_copy(acc2.at[pl.ds(s * ZCH1, ZCH1)], ztmp)
    pltpu.sync_copy(ztmp, degp.at[pl.ds((c * 2 + 1) * NPAD1 + s * ZCH1, ZCH1)])


_deg_kernel = pl.kernel(
    _deg_body,
    out_type=jax.ShapeDtypeStruct((NC * 2 * NPAD1,), jnp.float32),
    mesh=_mesh,
    scratch_types=[
        pltpu.VMEM_SHARED((NPAD1,), jnp.float32),
        pltpu.VMEM_SHARED((NPAD1,), jnp.float32),
        pltpu.VMEM((IB1, CHUNK), jnp.int32),
        pltpu.VMEM((IB2, CHUNK), jnp.int32),
        pltpu.VMEM((CHUNK,), jnp.float32),
        pltpu.VMEM((ZCH1,), jnp.float32),
    ],
)


def _agg_body(ni, ib, u_hbm, spk, dpk, zrows, part, acc, evs, evd, rows_v,
              gsem):
    c = lax.axis_index("c")
    s = lax.axis_index("s")
    w = c * NS + s

    pltpu.sync_copy(zrows, rows_v)
    for j in range(4):
        pltpu.sync_copy(rows_v, acc.at[pl.ds(s * ZROWS + j * CHUNK, CHUNK)])
    tail = ZROWS - 4 * CHUNK
    pltpu.sync_copy(rows_v.at[pl.ds(0, tail)],
                    acc.at[pl.ds(s * ZROWS + 4 * CHUNK, tail)])
    plsc.subcore_barrier()

    ngr = ni // ib
    gbase = w * ngr

    def body(g, carry):
        pltpu.sync_copy(spk.at[gbase + g], evs)
        pltpu.sync_copy(dpk.at[gbase + g], evd)
        for j in range(ib):
            pltpu.async_copy(u_hbm.at[evs.at[j]], rows_v, gsem).wait()
            pltpu.sync_copy(rows_v, acc.at[evd.at[j]], add=True)
        return carry

    lax.fori_loop(0, ngr, body, 0)
    plsc.subcore_barrier()
    for j in range(4):
        pltpu.sync_copy(acc.at[pl.ds(s * ZROWS + j * CHUNK, CHUNK)], rows_v)
        pltpu.sync_copy(rows_v, part.at[c, pl.ds(s * ZROWS + j * CHUNK, CHUNK)])
    pltpu.sync_copy(acc.at[pl.ds(s * ZROWS + 4 * CHUNK, tail)],
                    rows_v.at[pl.ds(0, tail)])
    pltpu.sync_copy(rows_v.at[pl.ds(0, tail)],
                    part.at[c, pl.ds(s * ZROWS + 4 * CHUNK, tail)])


def _make_agg(ni, ib):
    assert ni % ib == 0
    return pl.kernel(
        functools.partial(_agg_body, ni, ib),
        out_type=jax.ShapeDtypeStruct((NC, NPAD2, F), jnp.float32),
        mesh=_mesh,
        scratch_types=[
            pltpu.VMEM_SHARED((NPAD2, F), jnp.float32),
            pltpu.VMEM((ib, CHUNK), jnp.int32),
            pltpu.VMEM((ib, CHUNK), jnp.int32),
            pltpu.VMEM((CHUNK, F), jnp.float32),
            pltpu.SemaphoreType.DMA,
        ],
    )


_agg1 = _make_agg(N1, IB1)
_agg2 = _make_agg(N2, IB2)


def _bn(x, g, b, eps=1e-5):
    mu = jnp.mean(x, axis=0, keepdims=True)
    var = jnp.mean((x - mu) ** 2, axis=0, keepdims=True)
    return (x - mu) / jnp.sqrt(var + eps) * g + b


def _lrelu(x):
    return jnp.where(x >= 0, x, 0.1 * x)


def _dis_cols(degp):
    degsum = degp[0] + degp[1]
    dis1 = lax.rsqrt(degsum[0, :N] + 1.0)[:, None]
    dis2 = lax.rsqrt(degsum[1, :N] + 1.0)[:, None]
    return dis1, dis2


def _tc1_body(x_ref, w1_ref, degp_ref, u1_ref, u2_ref):
    h1 = jnp.dot(x_ref[...], w1_ref[...], preferred_element_type=jnp.float32)
    dis1, dis2 = _dis_cols(degp_ref[...])
    u1_ref[...] = h1 * dis1
    u2_ref[...] = h1 * dis2


_tc1 = pl.pallas_call(
    _tc1_body,
    out_shape=[jax.ShapeDtypeStruct((N, F), jnp.float32),
               jax.ShapeDtypeStruct((N, F), jnp.float32)],
)


def _tc2_body(part2_ref, u2_ref, part1_ref, u1_ref, degp_ref, b1_ref, wt_ref,
              bt_ref, gz_ref, bz_ref, g1_ref, bb1_ref, w2_ref, v_ref):
    dis1, dis2 = _dis_cols(degp_ref[...])
    a2 = part2_ref[0, :N] + part2_ref[1, :N] + u2_ref[...]
    z0 = a2 * dis2 + b1_ref[...]
    z = _bn(jnp.dot(z0, wt_ref[...], preferred_element_type=jnp.float32)
            + bt_ref[...], gz_ref[...], bz_ref[...])
    a1 = part1_ref[0, :N] + part1_ref[1, :N] + u1_ref[...]
    g = a1 * dis1 + b1_ref[...]
    h = _lrelu(_bn(g + z, g1_ref[...], bb1_ref[...]))
    v_ref[...] = jnp.dot(h, w2_ref[...], preferred_element_type=jnp.float32) * dis1


_tc2 = pl.pallas_call(
    _tc2_body,
    out_shape=jax.ShapeDtypeStruct((N, F), jnp.float32),
)


def _tc3_body(part3_ref, v_ref, degp_ref, b2_ref, g2_ref, bb2_ref, p_ref,
              bd_ref, wg_ref, bg_ref, out_ref, yp_ref):
    dis1, _ = _dis_cols(degp_ref[...])
    a3 = part3_ref[0, :N] + part3_ref[1, :N] + v_ref[...]
    hh = a3 * dis1 + b2_ref[...]
    hh = _lrelu(_bn(hh, g2_ref[...], bb2_ref[...])) + bd_ref[0, 0]
    p = p_ref[0, 0]
    seg = N // NB
    rows = []
    for b in range(NB):
        blk = hh[b * seg:(b + 1) * seg]
        base = jnp.clip(blk, 0.0, 100.0) + 1e-12
        xc = jnp.exp(p * jnp.log(base))
        rows.append(jnp.mean(xc, axis=0, keepdims=True))
    m = jnp.concatenate(rows, axis=0)
    aggv = jnp.exp(jnp.log(jnp.clip(m, 0.0, 100.0) + 1e-12) / p)
    outv = jnp.dot(aggv, wg_ref[...], preferred_element_type=jnp.float32) \
        + bg_ref[...]
    out_ref[...] = outv
    mx = jnp.max(outv, axis=1, keepdims=True)
    ii = lax.broadcasted_iota(jnp.int32, (NB, C), 1)
    yp_ref[...] = jnp.min(jnp.where(outv >= mx, ii, C), axis=1)[None, :]


_tc3 = pl.pallas_call(
    _tc3_body,
    out_shape=[jax.ShapeDtypeStruct((NB, C), jnp.float32),
               jax.ShapeDtypeStruct((1, NB), jnp.int32)],
)


def _pack_edges(ei, epad, ib):
    e = ei.shape[1]
    padn = epad - e
    src = jnp.concatenate([ei[0], jnp.zeros((padn,), jnp.int32)])
    dst = jnp.concatenate(
        [ei[1], N + (jnp.arange(padn, dtype=jnp.int32) % (NPAD2 - N))])
    return (src.reshape(-1, ib, CHUNK), dst.reshape(-1, ib, CHUNK))


def kernel(x, edge_index, internal_edge_index, ppi_edge_index, batch_size,
           W1, b1, Wt, bt, W2, b2, gz, bz, g1, bb1, g2, bb2, p, Wg, bg):
    spk1, dpk1 = _pack_edges(edge_index, E1P, IB1)
    spk2, dpk2 = _pack_edges(internal_edge_index, E2P, IB2)
    zflat = jnp.zeros((ZCH1,), jnp.float32)
    zrows = jnp.zeros((CHUNK, F), jnp.float32)

    degp = _deg_kernel(dpk1, dpk2, zflat).reshape(NC, 2, NPAD1)
    u1, u2 = _tc1(x, W1, degp)
    part2 = _agg2(u2, spk2, dpk2, zrows)
    part1 = _agg1(u1, spk1, dpk1, zrows)
    v = _tc2(part2, u2, part1, u1, degp, b1, Wt, bt, gz, bz, g1, bb1, W2)
    part3 = _agg1(v, spk1, dpk1, zrows)

    pexp = p.reshape(1, 1)
    bd = (jnp.asarray(batch_size, jnp.float32) - jnp.float32(NB)).reshape(1, 1)
    out, yp = _tc3(part3, v, degp, b2, g2, bb2, pexp, bd, Wg, bg)
    return (out, yp.reshape(NB))


# speedup vs baseline: 1.7666x; 1.7666x over previous
import functools

import jax
import jax.numpy as jnp
from jax import lax
from jax.experimental import pallas as pl
from jax.experimental.pallas import tpu as pltpu
from jax.experimental.pallas import tpu_sc as plsc

N = 10000
F = 128
C = 10
NB = 8
NC, NS = 2, 16
NW = NC * NS
CHUNK = 128
NPAD1 = 10240
ZCH1 = NPAD1 // NS
NPAD2 = 10112
ZROWS = NPAD2 // NS
E1, E2 = 320000, 160000
_GRAN = NW * CHUNK
E1P = ((E1 + _GRAN - 1) // _GRAN) * _GRAN
E2P = ((E2 + _GRAN - 1) // _GRAN) * _GRAN
N1 = E1P // (NW * CHUNK)
N2 = E2P // (NW * CHUNK)

_mesh = plsc.VectorSubcoreMesh(core_axis_name="c", subcore_axis_name="s",
                               num_cores=NC, num_subcores=NS)


def _deg_body(dst1, dst2, zflat, degp, acc1, acc2, idx_v, ones_v, ztmp):
    c = lax.axis_index("c")
    s = lax.axis_index("s")
    w = c * NS + s
    pltpu.sync_copy(zflat, ztmp)
    pltpu.sync_copy(ztmp, acc1.at[pl.ds(s * ZCH1, ZCH1)])
    pltpu.sync_copy(ztmp, acc2.at[pl.ds(s * ZCH1, ZCH1)])
    for j in range(CHUNK // 16):
        ones_v[pl.ds(j * 16, 16)] = jnp.ones((16,), jnp.float32)
    plsc.subcore_barrier()

    def body1(i, carry):
        base = pl.multiple_of((w * N1 + i) * CHUNK, CHUNK)
        pltpu.sync_copy(dst1.at[pl.ds(base, CHUNK)], idx_v)
        pltpu.sync_copy(ones_v, acc1.at[idx_v], add=True)
        return carry

    def body2(i, carry):
        base = pl.multiple_of((w * N2 + i) * CHUNK, CHUNK)
        pltpu.sync_copy(dst2.at[pl.ds(base, CHUNK)], idx_v)
        pltpu.sync_copy(ones_v, acc2.at[idx_v], add=True)
        return carry

    lax.fori_loop(0, N1, body1, 0)
    lax.fori_loop(0, N2, body2, 0)
    plsc.subcore_barrier()
    pltpu.sync_copy(acc1.at[pl.ds(s * ZCH1, ZCH1)], ztmp)
    pltpu.sync_copy(ztmp, degp.at[pl.ds((c * 2 + 0) * NPAD1 + s * ZCH1, ZCH1)])
    pltpu.sync_copy(acc2.at[pl.ds(s * ZCH1, ZCH1)], ztmp)
    pltpu.sync_copy(ztmp, degp.at[pl.ds((c * 2 + 1) * NPAD1 + s * ZCH1, ZCH1)])


_deg_kernel = pl.kernel(
    _deg_body,
    out_type=jax.ShapeDtypeStruct((NC * 2 * NPAD1,), jnp.float32),
    mesh=_mesh,
    scratch_types=[
        pltpu.VMEM_SHARED((NPAD1,), jnp.float32),
        pltpu.VMEM_SHARED((NPAD1,), jnp.float32),
        pltpu.VMEM((CHUNK,), jnp.int32),
        pltpu.VMEM((CHUNK,), jnp.float32),
        pltpu.VMEM((ZCH1,), jnp.float32),
    ],
)


def _agg_body(ni, u_hbm, src_hbm, dst_hbm, zrows, part, acc, src_v, dst_v,
              rows_v, gsem):
    c = lax.axis_index("c")
    s = lax.axis_index("s")
    w = c * NS + s

    pltpu.sync_copy(zrows, rows_v)
    for j in range(4):
        pltpu.sync_copy(rows_v, acc.at[pl.ds(s * ZROWS + j * CHUNK, CHUNK)])
    tail = ZROWS - 4 * CHUNK
    pltpu.sync_copy(rows_v.at[pl.ds(0, tail)],
                    acc.at[pl.ds(s * ZROWS + 4 * CHUNK, tail)])
    plsc.subcore_barrier()

    def body(i, carry):
        base = pl.multiple_of((w * ni + i) * CHUNK, CHUNK)
        pltpu.sync_copy(src_hbm.at[pl.ds(base, CHUNK)], src_v)
        pltpu.sync_copy(dst_hbm.at[pl.ds(base, CHUNK)], dst_v)
        pltpu.async_copy(u_hbm.at[src_v], rows_v, gsem).wait()
        pltpu.sync_copy(rows_v, acc.at[dst_v], add=True)
        return carry

    lax.fori_loop(0, ni, body, 0)
    plsc.subcore_barrier()
    for j in range(4):
        pltpu.sync_copy(acc.at[pl.ds(s * ZROWS + j * CHUNK, CHUNK)], rows_v)
        pltpu.sync_copy(rows_v, part.at[c, pl.ds(s * ZROWS + j * CHUNK, CHUNK)])
    pltpu.sync_copy(acc.at[pl.ds(s * ZROWS + 4 * CHUNK, tail)],
                    rows_v.at[pl.ds(0, tail)])
    pltpu.sync_copy(rows_v.at[pl.ds(0, tail)],
                    part.at[c, pl.ds(s * ZROWS + 4 * CHUNK, tail)])


def _make_agg(ni):
    return pl.kernel(
        functools.partial(_agg_body, ni),
        out_type=jax.ShapeDtypeStruct((NC, NPAD2, F), jnp.float32),
        mesh=_mesh,
        scratch_types=[
            pltpu.VMEM_SHARED((NPAD2, F), jnp.float32),
            pltpu.VMEM((CHUNK,), jnp.int32),
            pltpu.VMEM((CHUNK,), jnp.int32),
            pltpu.VMEM((CHUNK, F), jnp.float32),
            pltpu.SemaphoreType.DMA,
        ],
    )


_agg1 = _make_agg(N1)
_agg2 = _make_agg(N2)


def _bn(x, g, b, eps=1e-5):
    mu = jnp.mean(x, axis=0, keepdims=True)
    var = jnp.mean((x - mu) ** 2, axis=0, keepdims=True)
    return (x - mu) / jnp.sqrt(var + eps) * g + b


def _lrelu(x):
    return jnp.where(x >= 0, x, 0.1 * x)


def _dis_cols(degp):
    degsum = degp[0] + degp[1]
    dis1 = lax.rsqrt(degsum[0, :N] + 1.0)[:, None]
    dis2 = lax.rsqrt(degsum[1, :N] + 1.0)[:, None]
    return dis1, dis2


def _tcmm_body(x_ref, w1_ref, h1_ref):
    h1_ref[...] = jnp.dot(x_ref[...], w1_ref[...],
                          preferred_element_type=jnp.float32)


_tcmm = pl.pallas_call(
    _tcmm_body,
    out_shape=jax.ShapeDtypeStruct((N, F), jnp.float32),
)


def _tc1_body(h1_ref, degp_ref, u1_ref, u2_ref):
    h1 = h1_ref[...]
    dis1, dis2 = _dis_cols(degp_ref[...])
    u1_ref[...] = h1 * dis1
    u2_ref[...] = h1 * dis2


_tc1 = pl.pallas_call(
    _tc1_body,
    out_shape=[jax.ShapeDtypeStruct((N, F), jnp.float32),
               jax.ShapeDtypeStruct((N, F), jnp.float32)],
)


def _tc2a_body(part2_ref, u2_ref, degp_ref, b1_ref, wt_ref, bt_ref, gz_ref,
               bz_ref, z_ref):
    _, dis2 = _dis_cols(degp_ref[...])
    a2 = part2_ref[0, :N] + part2_ref[1, :N] + u2_ref[...]
    z0 = a2 * dis2 + b1_ref[...]
    z_ref[...] = _bn(jnp.dot(z0, wt_ref[...], preferred_element_type=jnp.float32)
                     + bt_ref[...], gz_ref[...], bz_ref[...])


_tc2a = pl.pallas_call(
    _tc2a_body,
    out_shape=jax.ShapeDtypeStruct((N, F), jnp.float32),
)


def _tc2b_body(part1_ref, u1_ref, z_ref, degp_ref, b1_ref, g1_ref, bb1_ref,
               w2_ref, v_ref):
    dis1, _ = _dis_cols(degp_ref[...])
    a1 = part1_ref[0, :N] + part1_ref[1, :N] + u1_ref[...]
    g = a1 * dis1 + b1_ref[...]
    h = _lrelu(_bn(g + z_ref[...], g1_ref[...], bb1_ref[...]))
    v_ref[...] = jnp.dot(h, w2_ref[...], preferred_element_type=jnp.float32) * dis1


_tc2b = pl.pallas_call(
    _tc2b_body,
    out_shape=jax.ShapeDtypeStruct((N, F), jnp.float32),
)


def _tc3_body(part3_ref, v_ref, degp_ref, b2_ref, g2_ref, bb2_ref, p_ref,
              bd_ref, wg_ref, bg_ref, out_ref, yp_ref):
    dis1, _ = _dis_cols(degp_ref[...])
    a3 = part3_ref[0, :N] + part3_ref[1, :N] + v_ref[...]
    hh = a3 * dis1 + b2_ref[...]
    hh = _lrelu(_bn(hh, g2_ref[...], bb2_ref[...])) + bd_ref[0, 0]
    p = p_ref[0, 0]
    seg = N // NB
    rows = []
    for b in range(NB):
        blk = hh[b * seg:(b + 1) * seg]
        base = jnp.clip(blk, 0.0, 100.0) + 1e-12
        xc = jnp.exp(p * jnp.log(base))
        rows.append(jnp.mean(xc, axis=0, keepdims=True))
    m = jnp.concatenate(rows, axis=0)
    aggv = jnp.exp(jnp.log(jnp.clip(m, 0.0, 100.0) + 1e-12) / p)
    outv = jnp.dot(aggv, wg_ref[...], preferred_element_type=jnp.float32) \
        + bg_ref[...]
    out_ref[...] = outv
    mx = jnp.max(outv, axis=1, keepdims=True)
    ii = lax.broadcasted_iota(jnp.int32, (NB, C), 1)
    yp_ref[...] = jnp.min(jnp.where(outv >= mx, ii, C), axis=1)[None, :]


_tc3 = pl.pallas_call(
    _tc3_body,
    out_shape=[jax.ShapeDtypeStruct((NB, C), jnp.float32),
               jax.ShapeDtypeStruct((1, NB), jnp.int32)],
)


def _pad_edges(ei, epad):
    e = ei.shape[1]
    padn = epad - e
    src = jnp.concatenate([ei[0], jnp.zeros((padn,), jnp.int32)])
    dst = jnp.concatenate(
        [ei[1], N + (jnp.arange(padn, dtype=jnp.int32) % (NPAD2 - N))])
    return src, dst


def kernel(x, edge_index, internal_edge_index, ppi_edge_index, batch_size,
           W1, b1, Wt, bt, W2, b2, gz, bz, g1, bb1, g2, bb2, p, Wg, bg):
    src1, dst1 = _pad_edges(edge_index, E1P)
    src2, dst2 = _pad_edges(internal_edge_index, E2P)
    zflat = jnp.zeros((ZCH1,), jnp.float32)
    zrows = jnp.zeros((CHUNK, F), jnp.float32)

    h1 = _tcmm(x, W1)
    degp = _deg_kernel(dst1, dst2, zflat).reshape(NC, 2, NPAD1)
    u1, u2 = _tc1(h1, degp)
    part2 = _agg2(u2, src2, dst2, zrows)
    part1 = _agg1(u1, src1, dst1, zrows)
    z = _tc2a(part2, u2, degp, b1, Wt, bt, gz, bz)
    v = _tc2b(part1, u1, z, degp, b1, g1, bb1, W2)
    part3 = _agg1(v, src1, dst1, zrows)

    pexp = p.reshape(1, 1)
    bd = (jnp.asarray(batch_size, jnp.float32) - jnp.float32(NB)).reshape(1, 1)
    out, yp = _tc3(part3, v, degp, b2, g2, bb2, pexp, bd, Wg, bg)
    return (out, yp.reshape(NB))


# speedup vs baseline: 1.9213x; 1.0876x over previous
import functools

import jax
import jax.numpy as jnp
from jax import lax
from jax.experimental import pallas as pl
from jax.experimental.pallas import tpu as pltpu
from jax.experimental.pallas import tpu_sc as plsc

N = 10000
F = 128
C = 10
NB = 8
NC, NS = 2, 16
NW = NC * NS
CHUNK = 128
NPAD1 = 10240
ZCH1 = NPAD1 // NS
NPAD2 = 10112
ZROWS = NPAD2 // NS
E1, E2 = 320000, 160000
_GRAN = NW * CHUNK
E1P = ((E1 + _GRAN - 1) // _GRAN) * _GRAN
E2P = ((E2 + _GRAN - 1) // _GRAN) * _GRAN
N1 = E1P // (NW * CHUNK)
N2 = E2P // (NW * CHUNK)

_mesh = plsc.VectorSubcoreMesh(core_axis_name="c", subcore_axis_name="s",
                               num_cores=NC, num_subcores=NS)


def _deg_body(dst1, dst2, zflat, degp, acc1, acc2, idx_v, ones_v, ztmp):
    c = lax.axis_index("c")
    s = lax.axis_index("s")
    w = c * NS + s
    pltpu.sync_copy(zflat, ztmp)
    pltpu.sync_copy(ztmp, acc1.at[pl.ds(s * ZCH1, ZCH1)])
    pltpu.sync_copy(ztmp, acc2.at[pl.ds(s * ZCH1, ZCH1)])
    for j in range(CHUNK // 16):
        ones_v[pl.ds(j * 16, 16)] = jnp.ones((16,), jnp.float32)
    plsc.subcore_barrier()

    def body1(i, carry):
        base = pl.multiple_of((w * N1 + i) * CHUNK, CHUNK)
        pltpu.sync_copy(dst1.at[pl.ds(base, CHUNK)], idx_v)
        pltpu.sync_copy(ones_v, acc1.at[idx_v], add=True)
        return carry

    def body2(i, carry):
        base = pl.multiple_of((w * N2 + i) * CHUNK, CHUNK)
        pltpu.sync_copy(dst2.at[pl.ds(base, CHUNK)], idx_v)
        pltpu.sync_copy(ones_v, acc2.at[idx_v], add=True)
        return carry

    lax.fori_loop(0, N1, body1, 0)
    lax.fori_loop(0, N2, body2, 0)
    plsc.subcore_barrier()
    pltpu.sync_copy(acc1.at[pl.ds(s * ZCH1, ZCH1)], ztmp)
    pltpu.sync_copy(ztmp, degp.at[pl.ds((c * 2 + 0) * NPAD1 + s * ZCH1, ZCH1)])
    pltpu.sync_copy(acc2.at[pl.ds(s * ZCH1, ZCH1)], ztmp)
    pltpu.sync_copy(ztmp, degp.at[pl.ds((c * 2 + 1) * NPAD1 + s * ZCH1, ZCH1)])


_deg_kernel = pl.kernel(
    _deg_body,
    out_type=jax.ShapeDtypeStruct((NC * 2 * NPAD1,), jnp.float32),
    mesh=_mesh,
    scratch_types=[
        pltpu.VMEM_SHARED((NPAD1,), jnp.float32),
        pltpu.VMEM_SHARED((NPAD1,), jnp.float32),
        pltpu.VMEM((CHUNK,), jnp.int32),
        pltpu.VMEM((CHUNK,), jnp.float32),
        pltpu.VMEM((ZCH1,), jnp.float32),
    ],
)


def _agg_body(ni0, ni1, u_hbm, src_hbm, dst_hbm, zrows, part, acc, src_v,
              dst_v, rows_v, gsem):
    c = lax.axis_index("c")
    s = lax.axis_index("s")
    cbase = jnp.where(c == 0, s * ni0, NS * ni0 + s * ni1)
    ni = jnp.where(c == 0, ni0, ni1)

    pltpu.sync_copy(zrows, rows_v)
    for j in range(4):
        pltpu.sync_copy(rows_v, acc.at[pl.ds(s * ZROWS + j * CHUNK, CHUNK)])
    tail = ZROWS - 4 * CHUNK
    pltpu.sync_copy(rows_v.at[pl.ds(0, tail)],
                    acc.at[pl.ds(s * ZROWS + 4 * CHUNK, tail)])
    plsc.subcore_barrier()

    def body(i, carry):
        base = pl.multiple_of((cbase + i) * CHUNK, CHUNK)
        pltpu.sync_copy(src_hbm.at[pl.ds(base, CHUNK)], src_v)
        pltpu.sync_copy(dst_hbm.at[pl.ds(base, CHUNK)], dst_v)
        pltpu.async_copy(u_hbm.at[src_v], rows_v, gsem).wait()
        pltpu.sync_copy(rows_v, acc.at[dst_v], add=True)
        return carry

    lax.fori_loop(0, ni, body, 0)
    plsc.subcore_barrier()
    for j in range(4):
        pltpu.sync_copy(acc.at[pl.ds(s * ZROWS + j * CHUNK, CHUNK)], rows_v)
        pltpu.sync_copy(rows_v, part.at[c, pl.ds(s * ZROWS + j * CHUNK, CHUNK)])
    pltpu.sync_copy(acc.at[pl.ds(s * ZROWS + 4 * CHUNK, tail)],
                    rows_v.at[pl.ds(0, tail)])
    pltpu.sync_copy(rows_v.at[pl.ds(0, tail)],
                    part.at[c, pl.ds(s * ZROWS + 4 * CHUNK, tail)])


def _make_agg(ni0, ni1):
    return pl.kernel(
        functools.partial(_agg_body, ni0, ni1),
        out_type=jax.ShapeDtypeStruct((NC, NPAD2, F), jnp.float32),
        mesh=_mesh,
        scratch_types=[
            pltpu.VMEM_SHARED((NPAD2, F), jnp.float32),
            pltpu.VMEM((CHUNK,), jnp.int32),
            pltpu.VMEM((CHUNK,), jnp.int32),
            pltpu.VMEM((CHUNK, F), jnp.float32),
            pltpu.SemaphoreType.DMA,
        ],
    )


NT1 = E1P // (NS * CHUNK)
NT2 = E2P // (NS * CHUNK)
NI1_0, NI1_1 = 95, NT1 - 95
NI2_0, NI2_1 = 48, NT2 - 48
_agg1 = _make_agg(NI1_0, NI1_1)
_agg2 = _make_agg(NI2_0, NI2_1)


def _bn(x, g, b, eps=1e-5):
    mu = jnp.mean(x, axis=0, keepdims=True)
    var = jnp.mean((x - mu) ** 2, axis=0, keepdims=True)
    return (x - mu) / jnp.sqrt(var + eps) * g + b


def _lrelu(x):
    return jnp.where(x >= 0, x, 0.1 * x)


def _dis_cols(degp):
    degsum = degp[0] + degp[1]
    dis1 = lax.rsqrt(degsum[0, :N] + 1.0)[:, None]
    dis2 = lax.rsqrt(degsum[1, :N] + 1.0)[:, None]
    return dis1, dis2


def _tcmm_body(x_ref, w1_ref, h1_ref):
    h1_ref[...] = jnp.dot(x_ref[...], w1_ref[...],
                          preferred_element_type=jnp.float32)


_tcmm = pl.pallas_call(
    _tcmm_body,
    out_shape=jax.ShapeDtypeStruct((N, F), jnp.float32),
)


def _tc1_body(h1_ref, degp_ref, u1_ref, u2_ref):
    h1 = h1_ref[...]
    dis1, dis2 = _dis_cols(degp_ref[...])
    u1_ref[...] = h1 * dis1
    u2_ref[...] = h1 * dis2


_tc1 = pl.pallas_call(
    _tc1_body,
    out_shape=[jax.ShapeDtypeStruct((N, F), jnp.float32),
               jax.ShapeDtypeStruct((N, F), jnp.float32)],
)


def _tc2a_body(part2_ref, u2_ref, degp_ref, b1_ref, wt_ref, bt_ref, gz_ref,
               bz_ref, z_ref):
    _, dis2 = _dis_cols(degp_ref[...])
    a2 = part2_ref[0, :N] + part2_ref[1, :N] + u2_ref[...]
    z0 = a2 * dis2 + b1_ref[...]
    z_ref[...] = _bn(jnp.dot(z0, wt_ref[...], preferred_element_type=jnp.float32)
                     + bt_ref[...], gz_ref[...], bz_ref[...])


_tc2a = pl.pallas_call(
    _tc2a_body,
    out_shape=jax.ShapeDtypeStruct((N, F), jnp.float32),
)


def _tc2b_body(part1_ref, u1_ref, z_ref, degp_ref, b1_ref, g1_ref, bb1_ref,
               w2_ref, v_ref):
    dis1, _ = _dis_cols(degp_ref[...])
    a1 = part1_ref[0, :N] + part1_ref[1, :N] + u1_ref[...]
    g = a1 * dis1 + b1_ref[...]
    h = _lrelu(_bn(g + z_ref[...], g1_ref[...], bb1_ref[...]))
    v_ref[...] = jnp.dot(h, w2_ref[...], preferred_element_type=jnp.float32) * dis1


_tc2b = pl.pallas_call(
    _tc2b_body,
    out_shape=jax.ShapeDtypeStruct((N, F), jnp.float32),
)


def _tc3_body(part3_ref, v_ref, degp_ref, b2_ref, g2_ref, bb2_ref, p_ref,
              bd_ref, wg_ref, bg_ref, out_ref, yp_ref):
    dis1, _ = _dis_cols(degp_ref[...])
    a3 = part3_ref[0, :N] + part3_ref[1, :N] + v_ref[...]
    hh = a3 * dis1 + b2_ref[...]
    hh = _lrelu(_bn(hh, g2_ref[...], bb2_ref[...])) + bd_ref[0, 0]
    p = p_ref[0, 0]
    seg = N // NB
    rows = []
    for b in range(NB):
        blk = hh[b * seg:(b + 1) * seg]
        base = jnp.clip(blk, 0.0, 100.0) + 1e-12
        xc = jnp.exp(p * jnp.log(base))
        rows.append(jnp.mean(xc, axis=0, keepdims=True))
    m = jnp.concatenate(rows, axis=0)
    aggv = jnp.exp(jnp.log(jnp.clip(m, 0.0, 100.0) + 1e-12) / p)
    outv = jnp.dot(aggv, wg_ref[...], preferred_element_type=jnp.float32) \
        + bg_ref[...]
    out_ref[...] = outv
    mx = jnp.max(outv, axis=1, keepdims=True)
    ii = lax.broadcasted_iota(jnp.int32, (NB, C), 1)
    yp_ref[...] = jnp.min(jnp.where(outv >= mx, ii, C), axis=1)[None, :]


_tc3 = pl.pallas_call(
    _tc3_body,
    out_shape=[jax.ShapeDtypeStruct((NB, C), jnp.float32),
               jax.ShapeDtypeStruct((1, NB), jnp.int32)],
)


def _pad_edges(ei, epad):
    e = ei.shape[1]
    padn = epad - e
    src = jnp.concatenate([ei[0], jnp.zeros((padn,), jnp.int32)])
    dst = jnp.concatenate(
        [ei[1], N + (jnp.arange(padn, dtype=jnp.int32) % (NPAD2 - N))])
    return src, dst


def kernel(x, edge_index, internal_edge_index, ppi_edge_index, batch_size,
           W1, b1, Wt, bt, W2, b2, gz, bz, g1, bb1, g2, bb2, p, Wg, bg):
    src1, dst1 = _pad_edges(edge_index, E1P)
    src2, dst2 = _pad_edges(internal_edge_index, E2P)
    zflat = jnp.zeros((ZCH1,), jnp.float32)
    zrows = jnp.zeros((CHUNK, F), jnp.float32)

    h1 = _tcmm(x, W1)
    degp = _deg_kernel(dst1, dst2, zflat).reshape(NC, 2, NPAD1)
    u1, u2 = _tc1(h1, degp)
    part2 = _agg2(u2, src2, dst2, zrows)
    part1 = _agg1(u1, src1, dst1, zrows)
    z = _tc2a(part2, u2, degp, b1, Wt, bt, gz, bz)
    v = _tc2b(part1, u1, z, degp, b1, g1, bb1, W2)
    part3 = _agg1(v, src1, dst1, zrows)

    pexp = p.reshape(1, 1)
    bd = (jnp.asarray(batch_size, jnp.float32) - jnp.float32(NB)).reshape(1, 1)
    out, yp = _tc3(part3, v, degp, b2, g2, bb2, pexp, bd, Wg, bg)
    return (out, yp.reshape(NB))


# speedup vs baseline: 2.0178x; 1.0502x over previous
import functools

import jax
import jax.numpy as jnp
from jax import lax
from jax.experimental import pallas as pl
from jax.experimental.pallas import tpu as pltpu
from jax.experimental.pallas import tpu_sc as plsc

N = 10000
F = 128
C = 10
NB = 8
NC, NS = 2, 16
NW = NC * NS
CHUNK = 128
NPAD1 = 10240
ZCH1 = NPAD1 // NS
NPAD2 = 10112
ZROWS = NPAD2 // NS
E1, E2 = 320000, 160000
_GRAN = NW * CHUNK
E1P = ((E1 + _GRAN - 1) // _GRAN) * _GRAN
E2P = ((E2 + _GRAN - 1) // _GRAN) * _GRAN
N1 = E1P // (NW * CHUNK)
N2 = E2P // (NW * CHUNK)

_mesh = plsc.VectorSubcoreMesh(core_axis_name="c", subcore_axis_name="s",
                               num_cores=NC, num_subcores=NS)


def _deg_body(dst1, dst2, zflat, degp, acc1, acc2, idx_v, ones_v, ztmp):
    c = lax.axis_index("c")
    s = lax.axis_index("s")
    w = c * NS + s
    pltpu.sync_copy(zflat, ztmp)
    pltpu.sync_copy(ztmp, acc1.at[pl.ds(s * ZCH1, ZCH1)])
    pltpu.sync_copy(ztmp, acc2.at[pl.ds(s * ZCH1, ZCH1)])
    for j in range(CHUNK // 16):
        ones_v[pl.ds(j * 16, 16)] = jnp.ones((16,), jnp.float32)
    plsc.subcore_barrier()

    def body1(i, carry):
        base = pl.multiple_of((w * N1 + i) * CHUNK, CHUNK)
        pltpu.sync_copy(dst1.at[pl.ds(base, CHUNK)], idx_v)
        pltpu.sync_copy(ones_v, acc1.at[idx_v], add=True)
        return carry

    def body2(i, carry):
        base = pl.multiple_of((w * N2 + i) * CHUNK, CHUNK)
        pltpu.sync_copy(dst2.at[pl.ds(base, CHUNK)], idx_v)
        pltpu.sync_copy(ones_v, acc2.at[idx_v], add=True)
        return carry

    lax.fori_loop(0, N1, body1, 0)
    lax.fori_loop(0, N2, body2, 0)
    plsc.subcore_barrier()
    pltpu.sync_copy(acc1.at[pl.ds(s * ZCH1, ZCH1)], ztmp)
    pltpu.sync_copy(ztmp, degp.at[pl.ds((c * 2 + 0) * NPAD1 + s * ZCH1, ZCH1)])
    pltpu.sync_copy(acc2.at[pl.ds(s * ZCH1, ZCH1)], ztmp)
    pltpu.sync_copy(ztmp, degp.at[pl.ds((c * 2 + 1) * NPAD1 + s * ZCH1, ZCH1)])


_deg_kernel = pl.kernel(
    _deg_body,
    out_type=jax.ShapeDtypeStruct((NC * 2 * NPAD1,), jnp.float32),
    mesh=_mesh,
    scratch_types=[
        pltpu.VMEM_SHARED((NPAD1,), jnp.float32),
        pltpu.VMEM_SHARED((NPAD1,), jnp.float32),
        pltpu.VMEM((CHUNK,), jnp.int32),
        pltpu.VMEM((CHUNK,), jnp.float32),
        pltpu.VMEM((ZCH1,), jnp.float32),
    ],
)


def _agg_body(ni0, ni1, u_hbm, src_hbm, dst_hbm, zrows, part, acc, src_v,
              dst_v, rows_v, gsem):
    c = lax.axis_index("c")
    s = lax.axis_index("s")
    cbase = jnp.where(c == 0, s * ni0, NS * ni0 + s * ni1)
    ni = jnp.where(c == 0, ni0, ni1)

    pltpu.sync_copy(zrows, rows_v)
    for j in range(4):
        pltpu.sync_copy(rows_v, acc.at[pl.ds(s * ZROWS + j * CHUNK, CHUNK)])
    tail = ZROWS - 4 * CHUNK
    pltpu.sync_copy(rows_v.at[pl.ds(0, tail)],
                    acc.at[pl.ds(s * ZROWS + 4 * CHUNK, tail)])
    plsc.subcore_barrier()

    def body(i, carry):
        base = pl.multiple_of((cbase + i) * CHUNK, CHUNK)
        pltpu.sync_copy(src_hbm.at[pl.ds(base, CHUNK)], src_v)
        pltpu.sync_copy(dst_hbm.at[pl.ds(base, CHUNK)], dst_v)
        pltpu.async_copy(u_hbm.at[src_v], rows_v, gsem).wait()
        pltpu.sync_copy(rows_v, acc.at[dst_v], add=True)
        return carry

    lax.fori_loop(0, ni, body, 0)
    plsc.subcore_barrier()
    for j in range(4):
        pltpu.sync_copy(acc.at[pl.ds(s * ZROWS + j * CHUNK, CHUNK)], rows_v)
        pltpu.sync_copy(rows_v, part.at[c, pl.ds(s * ZROWS + j * CHUNK, CHUNK)])
    pltpu.sync_copy(acc.at[pl.ds(s * ZROWS + 4 * CHUNK, tail)],
                    rows_v.at[pl.ds(0, tail)])
    pltpu.sync_copy(rows_v.at[pl.ds(0, tail)],
                    part.at[c, pl.ds(s * ZROWS + 4 * CHUNK, tail)])


def _make_agg(ni0, ni1):
    return pl.kernel(
        functools.partial(_agg_body, ni0, ni1),
        out_type=jax.ShapeDtypeStruct((NC, NPAD2, F), jnp.float32),
        mesh=_mesh,
        scratch_types=[
            pltpu.VMEM_SHARED((NPAD2, F), jnp.float32),
            pltpu.VMEM((CHUNK,), jnp.int32),
            pltpu.VMEM((CHUNK,), jnp.int32),
            pltpu.VMEM((CHUNK, F), jnp.float32),
            pltpu.SemaphoreType.DMA,
        ],
    )


NT1 = E1P // (NS * CHUNK)
NT2 = E2P // (NS * CHUNK)
NI1_0, NI1_1 = 112, NT1 - 112
NI2_0, NI2_1 = 57, NT2 - 57
_agg1 = _make_agg(NI1_0, NI1_1)
_agg2 = _make_agg(NI2_0, NI2_1)


def _bn(x, g, b, eps=1e-5):
    mu = jnp.mean(x, axis=0, keepdims=True)
    var = jnp.mean((x - mu) ** 2, axis=0, keepdims=True)
    return (x - mu) / jnp.sqrt(var + eps) * g + b


def _lrelu(x):
    return jnp.where(x >= 0, x, 0.1 * x)


def _dis_cols(degp):
    degsum = degp[0] + degp[1]
    dis1 = lax.rsqrt(degsum[0, :N] + 1.0)[:, None]
    dis2 = lax.rsqrt(degsum[1, :N] + 1.0)[:, None]
    return dis1, dis2


def _tcmm_body(x_ref, w1_ref, h1_ref):
    h1_ref[...] = jnp.dot(x_ref[...], w1_ref[...],
                          preferred_element_type=jnp.float32)


_tcmm = pl.pallas_call(
    _tcmm_body,
    out_shape=jax.ShapeDtypeStruct((N, F), jnp.float32),
)


def _tc1_body(h1_ref, degp_ref, u1_ref, u2_ref):
    h1 = h1_ref[...]
    dis1, dis2 = _dis_cols(degp_ref[...])
    u1_ref[...] = h1 * dis1
    u2_ref[...] = h1 * dis2


_tc1 = pl.pallas_call(
    _tc1_body,
    out_shape=[jax.ShapeDtypeStruct((N, F), jnp.float32),
               jax.ShapeDtypeStruct((N, F), jnp.float32)],
)


def _tc2a_body(part2_ref, u2_ref, degp_ref, b1_ref, wt_ref, bt_ref, gz_ref,
               bz_ref, z_ref):
    _, dis2 = _dis_cols(degp_ref[...])
    a2 = part2_ref[0, :N] + part2_ref[1, :N] + u2_ref[...]
    z0 = a2 * dis2 + b1_ref[...]
    z_ref[...] = _bn(jnp.dot(z0, wt_ref[...], preferred_element_type=jnp.float32)
                     + bt_ref[...], gz_ref[...], bz_ref[...])


_tc2a = pl.pallas_call(
    _tc2a_body,
    out_shape=jax.ShapeDtypeStruct((N, F), jnp.float32),
)


def _tc2b_body(part1_ref, u1_ref, z_ref, degp_ref, b1_ref, g1_ref, bb1_ref,
               w2_ref, v_ref):
    dis1, _ = _dis_cols(degp_ref[...])
    a1 = part1_ref[0, :N] + part1_ref[1, :N] + u1_ref[...]
    g = a1 * dis1 + b1_ref[...]
    h = _lrelu(_bn(g + z_ref[...], g1_ref[...], bb1_ref[...]))
    v_ref[...] = jnp.dot(h, w2_ref[...], preferred_element_type=jnp.float32) * dis1


_tc2b = pl.pallas_call(
    _tc2b_body,
    out_shape=jax.ShapeDtypeStruct((N, F), jnp.float32),
)


def _tc3_body(part3_ref, v_ref, degp_ref, b2_ref, g2_ref, bb2_ref, p_ref,
              bd_ref, wg_ref, bg_ref, out_ref, yp_ref):
    dis1, _ = _dis_cols(degp_ref[...])
    a3 = part3_ref[0, :N] + part3_ref[1, :N] + v_ref[...]
    hh = a3 * dis1 + b2_ref[...]
    hh = _lrelu(_bn(hh, g2_ref[...], bb2_ref[...])) + bd_ref[0, 0]
    p = p_ref[0, 0]
    seg = N // NB
    rows = []
    for b in range(NB):
        blk = hh[b * seg:(b + 1) * seg]
        base = jnp.clip(blk, 0.0, 100.0) + 1e-12
        xc = jnp.exp(p * jnp.log(base))
        rows.append(jnp.mean(xc, axis=0, keepdims=True))
    m = jnp.concatenate(rows, axis=0)
    aggv = jnp.exp(jnp.log(jnp.clip(m, 0.0, 100.0) + 1e-12) / p)
    outv = jnp.dot(aggv, wg_ref[...], preferred_element_type=jnp.float32) \
        + bg_ref[...]
    out_ref[...] = outv
    mx = jnp.max(outv, axis=1, keepdims=True)
    ii = lax.broadcasted_iota(jnp.int32, (NB, C), 1)
    yp_ref[...] = jnp.min(jnp.where(outv >= mx, ii, C), axis=1)[None, :]


_tc3 = pl.pallas_call(
    _tc3_body,
    out_shape=[jax.ShapeDtypeStruct((NB, C), jnp.float32),
               jax.ShapeDtypeStruct((1, NB), jnp.int32)],
)


def _pad_edges(ei, epad):
    e = ei.shape[1]
    padn = epad - e
    src = jnp.concatenate([ei[0], jnp.zeros((padn,), jnp.int32)])
    dst = jnp.concatenate(
        [ei[1], N + (jnp.arange(padn, dtype=jnp.int32) % (NPAD2 - N))])
    return src, dst


def kernel(x, edge_index, internal_edge_index, ppi_edge_index, batch_size,
           W1, b1, Wt, bt, W2, b2, gz, bz, g1, bb1, g2, bb2, p, Wg, bg):
    src1, dst1 = _pad_edges(edge_index, E1P)
    src2, dst2 = _pad_edges(internal_edge_index, E2P)
    zflat = jnp.zeros((ZCH1,), jnp.float32)
    zrows = jnp.zeros((CHUNK, F), jnp.float32)

    h1 = _tcmm(x, W1)
    degp = _deg_kernel(dst1, dst2, zflat).reshape(NC, 2, NPAD1)
    u1, u2 = _tc1(h1, degp)
    part2 = _agg2(u2, src2, dst2, zrows)
    part1 = _agg1(u1, src1, dst1, zrows)
    z = _tc2a(part2, u2, degp, b1, Wt, bt, gz, bz)
    v = _tc2b(part1, u1, z, degp, b1, g1, bb1, W2)
    part3 = _agg1(v, src1, dst1, zrows)

    pexp = p.reshape(1, 1)
    bd = (jnp.asarray(batch_size, jnp.float32) - jnp.float32(NB)).reshape(1, 1)
    out, yp = _tc3(part3, v, degp, b2, g2, bb2, pexp, bd, Wg, bg)
    return (out, yp.reshape(NB))
